# Initial kernel scaffold; baseline (speedup 1.0000x reference)
import jax
import jax.numpy as jnp
from jax.experimental import pallas as pl


def kernel(x, edge_index, W1, b1, Wg_type, bg_type, Wl_type, bl_type, Wg_school, bg_school, Wl_school, bl_school, Wg_time, bg_time, Wl_time, bl_time, Wg_author, bg_author, Wl_author, bl_author):
    raise NotImplementedError("write your pallas kernel here")


# speedup vs baseline: 10.2241x; 10.2241x over previous
import functools

import jax
import jax.numpy as jnp
from jax import lax
from jax.experimental import pallas as pl
from jax.experimental.pallas import tpu as pltpu
from jax.experimental.pallas import tpu_sc as plsc

NC = 2
NS = 16
CHUNK = 128


def _sc_degree(n_pad, e_pad):
    ept = e_pad // (NC * NS)
    nch = ept // CHUNK
    rpt = n_pad // NS
    mesh = plsc.VectorSubcoreMesh(core_axis_name="c", subcore_axis_name="s")

    @functools.partial(
        pl.kernel,
        out_type=jax.ShapeDtypeStruct((NC * n_pad, 128), jnp.float32),
        mesh=mesh,
        scratch_types=[
            pltpu.VMEM_SHARED((n_pad, 128), jnp.float32),
            pltpu.VMEM((CHUNK,), jnp.int32),
            pltpu.VMEM((CHUNK, 128), jnp.float32),
        ],
    )
    def k(dst_hbm, zeros_hbm, ones_hbm, out_hbm, acc, dst_v, ones_v):
        c = lax.axis_index("c")
        s = lax.axis_index("s")
        r0 = s * rpt
        pltpu.sync_copy(zeros_hbm.at[pl.ds(r0, rpt)], acc.at[pl.ds(r0, rpt)])
        pltpu.sync_copy(ones_hbm, ones_v)
        plsc.subcore_barrier()
        base = (c * NS + s) * ept

        def body(i, carry):
            eb = base + i * CHUNK
            pltpu.sync_copy(dst_hbm.at[pl.ds(eb, CHUNK)], dst_v)
            pltpu.sync_copy(ones_v, acc.at[dst_v], add=True)
            return carry

        lax.fori_loop(0, nch, body, 0)
        plsc.subcore_barrier()
        pltpu.sync_copy(acc.at[pl.ds(r0, rpt)],
                        out_hbm.at[pl.ds(c * n_pad + r0, rpt)])

    return k


def _sc_scatter(n_pad, d2, e_pad):
    ept = e_pad // NS
    nch = ept // CHUNK
    rpt = n_pad // NS
    mesh = plsc.VectorSubcoreMesh(core_axis_name="c", subcore_axis_name="s")

    @functools.partial(
        pl.kernel,
        out_type=jax.ShapeDtypeStruct((NC * n_pad, d2), jnp.float32),
        mesh=mesh,
        scratch_types=[
            pltpu.VMEM_SHARED((n_pad, d2), jnp.float32),
            pltpu.VMEM((CHUNK,), jnp.int32),
            pltpu.VMEM((CHUNK,), jnp.int32),
            pltpu.VMEM((CHUNK, d2), jnp.float32),
            pltpu.SemaphoreType.DMA,
        ],
    )
    def k(y_hbm, src_hbm, dst_hbm, zeros_hbm, out_hbm,
          acc, src_v, dst_v, rows_v, sem):
        c = lax.axis_index("c")
        s = lax.axis_index("s")
        r0 = s * rpt
        pltpu.sync_copy(zeros_hbm.at[pl.ds(r0, rpt)], acc.at[pl.ds(r0, rpt)])
        plsc.subcore_barrier()
        base = c * e_pad + s * ept
        baseD = s * ept

        def body(i, carry):
            pltpu.sync_copy(src_hbm.at[pl.ds(base + i * CHUNK, CHUNK)], src_v)
            pltpu.sync_copy(dst_hbm.at[pl.ds(baseD + i * CHUNK, CHUNK)], dst_v)
            pltpu.async_copy(y_hbm.at[src_v], rows_v, sem).wait()
            pltpu.sync_copy(rows_v, acc.at[dst_v], add=True)
            return carry

        lax.fori_loop(0, nch, body, 0)
        plsc.subcore_barrier()
        pltpu.sync_copy(acc.at[pl.ds(r0, rpt)],
                        out_hbm.at[pl.ds(c * n_pad + r0, rpt)])

    return k


def _dis_from_hist(hist_blk):
    deg = 1.0 + hist_blk[0, :, 0] + hist_blk[1, :, 0]
    return lax.rsqrt(deg)


def _tc_first(x_p, W1, hist, n_pad, in_ch, hid):
    R = n_pad // 8
    h2 = hid // 2

    def body(x_ref, w_ref, hist_ref, y_ref):
        dis = _dis_from_hist(hist_ref)
        xw = jnp.dot(x_ref[...], w_ref[...], preferred_element_type=jnp.float32)
        y = xw * dis[:, None]
        y_ref[0] = y[:, :h2]
        y_ref[1] = y[:, h2:]

    return pl.pallas_call(
        body,
        grid=(n_pad // R,),
        in_specs=[
            pl.BlockSpec((R, in_ch), lambda i: (i, 0)),
            pl.BlockSpec((in_ch, hid), lambda i: (0, 0)),
            pl.BlockSpec((2, R, 128), lambda i: (0, i, 0)),
        ],
        out_specs=pl.BlockSpec((2, R, h2), lambda i: (0, i, 0)),
        out_shape=jax.ShapeDtypeStruct((2, n_pad, h2), jnp.float32),
    )(x_p, W1, hist)


def _tc_mid(s1, y1, hist, wg_cat, b1r, n_pad, hid, dg):
    R = n_pad // 8
    h2 = hid // 2
    g2 = dg // 2

    def body(s_ref, y_ref, hist_ref, w_ref, b_ref, o_ref):
        dis = _dis_from_hist(hist_ref)
        S = jnp.concatenate([s_ref[0], s_ref[1]], axis=1)
        Y = jnp.concatenate([y_ref[0], y_ref[1]], axis=1)
        h = jnp.maximum((S + Y) * dis[:, None] + b_ref[...], 0.0)
        y2 = jnp.dot(h, w_ref[...], preferred_element_type=jnp.float32)
        y2 = y2 * dis[:, None]
        o_ref[0] = y2[:, :g2]
        o_ref[1] = y2[:, g2:]

    return pl.pallas_call(
        body,
        grid=(n_pad // R,),
        in_specs=[
            pl.BlockSpec((2, R, h2), lambda i: (0, i, 0)),
            pl.BlockSpec((2, R, h2), lambda i: (0, i, 0)),
            pl.BlockSpec((2, R, 128), lambda i: (0, i, 0)),
            pl.BlockSpec((hid, dg), lambda i: (0, 0)),
            pl.BlockSpec((1, hid), lambda i: (0, 0)),
        ],
        out_specs=pl.BlockSpec((2, R, g2), lambda i: (0, i, 0)),
        out_shape=jax.ShapeDtypeStruct((2, n_pad, g2), jnp.float32),
    )(s1, y1, hist, wg_cat, b1r)


def _tc_heads(s2, y2, hist, wl_cat, bg_r, bl_r, n_pad, dg, zo):
    R = n_pad // 8
    g2 = dg // 2

    def body(s_ref, y_ref, hist_ref, w_ref, bg_ref, bl_ref, p_ref):
        dis = _dis_from_hist(hist_ref)
        S = jnp.concatenate([s_ref[0], s_ref[1]], axis=1)
        Y = jnp.concatenate([y_ref[0], y_ref[1]], axis=1)
        g = (S + Y) * dis[:, None] + bg_ref[...]
        z = jnp.dot(g, w_ref[...], preferred_element_type=jnp.float32)
        z = z + bl_ref[...]
        for k in range(zo // 128):
            zk = z[:, 128 * k:128 * (k + 1)]
            m = jnp.max(zk, axis=1, keepdims=True)
            e = jnp.exp(zk - m)
            p_ref[:, 128 * k:128 * (k + 1)] = e / jnp.sum(e, axis=1, keepdims=True)

    return pl.pallas_call(
        body,
        grid=(n_pad // R,),
        in_specs=[
            pl.BlockSpec((2, R, g2), lambda i: (0, i, 0)),
            pl.BlockSpec((2, R, g2), lambda i: (0, i, 0)),
            pl.BlockSpec((2, R, 128), lambda i: (0, i, 0)),
            pl.BlockSpec((dg, zo), lambda i: (0, 0)),
            pl.BlockSpec((1, dg), lambda i: (0, 0)),
            pl.BlockSpec((1, zo), lambda i: (0, 0)),
        ],
        out_specs=pl.BlockSpec((R, zo), lambda i: (i, 0)),
        out_shape=jax.ShapeDtypeStruct((n_pad, zo), jnp.float32),
    )(s2, y2, hist, wl_cat, bg_r, bl_r)


def kernel(x, edge_index, W1, b1,
           Wg_type, bg_type, Wl_type, bl_type,
           Wg_school, bg_school, Wl_school, bl_school,
           Wg_time, bg_time, Wl_time, bl_time,
           Wg_author, bg_author, Wl_author, bl_author):
    N, in_ch = x.shape
    E = edge_index.shape[1]
    hid = W1.shape[1]

    n_pad = ((N + 1 + 255) // 256) * 256
    egrp = NC * NS * CHUNK
    e_pad = ((E + egrp - 1) // egrp) * egrp

    src = edge_index[0]
    dst = edge_index[1]
    pad_e = e_pad - E
    src_p = jnp.concatenate([src, jnp.zeros((pad_e,), jnp.int32)])
    dst_p = jnp.concatenate([dst, jnp.full((pad_e,), N, jnp.int32)])
    src2 = jnp.concatenate([src_p, src_p + n_pad])

    x_p = jnp.pad(x, ((0, n_pad - N), (0, 0)))

    heads = [
        (Wg_type, bg_type, Wl_type, bl_type),
        (Wg_school, bg_school, Wl_school, bl_school),
        (Wg_time, bg_time, Wl_time, bl_time),
        (Wg_author, bg_author, Wl_author, bl_author),
    ]
    dims = [w.shape[1] for (w, _, _, _) in heads]
    offs = [0]
    for d in dims[:-1]:
        offs.append(offs[-1] + d)
    dg = ((offs[-1] + dims[-1] + 255) // 256) * 256
    zo = 128 * len(heads)

    wg_cat = jnp.zeros((hid, dg), jnp.float32)
    bg_cat = jnp.zeros((1, dg), jnp.float32)
    wl_cat = jnp.zeros((dg, zo), jnp.float32)
    bl_cat = jnp.full((1, zo), -1e30, jnp.float32)
    for k, (wg, bg, wl, bl) in enumerate(heads):
        d = dims[k]
        o = offs[k]
        wg_cat = wg_cat.at[:, o:o + d].set(wg)
        bg_cat = bg_cat.at[0, o:o + d].set(bg)
        wl_cat = wl_cat.at[o:o + d, 128 * k:128 * k + d].set(wl.T)
        bl_cat = bl_cat.at[0, 128 * k:128 * k + d].set(bl)

    zeros_h2 = jnp.zeros((n_pad, hid // 2), jnp.float32)
    ones128 = jnp.zeros((CHUNK, 128), jnp.float32).at[:, 0].set(1.0)
    zeros_g2 = zeros_h2 if dg == hid else jnp.zeros((n_pad, dg // 2), jnp.float32)

    hist = _sc_degree(n_pad, e_pad)(dst_p, zeros_h2, ones128)
    hist = hist.reshape(NC, n_pad, 128)

    y1 = _tc_first(x_p, W1, hist, n_pad, in_ch, hid)
    s1 = _sc_scatter(n_pad, hid // 2, e_pad)(
        y1.reshape(NC * n_pad, hid // 2), src2, dst_p, zeros_h2)
    s1 = s1.reshape(NC, n_pad, hid // 2)

    y2 = _tc_mid(s1, y1, hist, wg_cat, b1.reshape(1, hid), n_pad, hid, dg)
    s2 = _sc_scatter(n_pad, dg // 2, e_pad)(
        y2.reshape(NC * n_pad, dg // 2), src2, dst_p, zeros_g2)
    s2 = s2.reshape(NC, n_pad, dg // 2)

    P = _tc_heads(s2, y2, hist, wl_cat, bg_cat, bl_cat, n_pad, dg, zo)

    return tuple(P[:N, 128 * k:128 * k + dims[k]] for k in range(len(heads)))
